# Initial kernel scaffold; baseline (speedup 1.0000x reference)
import jax
import jax.numpy as jnp
from jax.experimental import pallas as pl


def kernel(data_in, pseudotimes_arr, ref_data, transform_mat, K):
    raise NotImplementedError("write your pallas kernel here")


# speedup vs baseline: 1.4725x; 1.4725x over previous
import functools

import jax
import jax.numpy as jnp
from jax import lax
from jax.experimental import pallas as pl
from jax.experimental.pallas import tpu as pltpu
from jax.experimental.pallas import tpu_sc as plsc

N_REF = 1_000_000
D_IN = 128
D_PC = 32
KTOP = 16
LANES = 16
NWORKERS = 32
BATCHES_TOTAL = N_REF // LANES
BASE_BATCHES = BATCHES_TOTAL // NWORKERS
EXTRA_BATCHES = BATCHES_TOTAL - BASE_BATCHES * NWORKERS
CHUNK_BATCHES = 63
CHUNK_ROWS = CHUNK_BATCHES * LANES
CHUNKS = BASE_BATCHES // CHUNK_BATCHES
ROWS_PER_WORKER = BASE_BATCHES * LANES
EXTRA_ROW0 = BASE_BATCHES * NWORKERS * LANES


def _merge_top16(td, tp, nd, np_):
    sd, sp = plsc.sort_key_val(nd, np_, descending=True)
    take = td <= sd
    ld = jnp.where(take, td, sd)
    lp = jnp.where(take, tp, sp)
    od, op = plsc.sort_key_val(ld, lp)
    return od, op


def _stage1_body(din_hbm, pt_hbm, ref_flat, tm_hbm, outd_hbm, outp_hbm,
                 din_v, tm_v, ref_v0, ref_v1, pt_v0, pt_v1, res_v):
    cid = lax.axis_index("c")
    sid = lax.axis_index("s")
    wid = cid * 16 + sid

    pltpu.sync_copy(din_hbm, din_v)
    pltpu.sync_copy(tm_hbm, tm_v)

    def qstep(j, qc):
        q0, q1 = qc
        dv = din_v[0, pl.ds(j * LANES, LANES)]
        for l in range(LANES):
            s = dv[l]
            row = j * LANES + l
            q0 = q0 + s * tm_v[row, pl.ds(0, 16)]
            q1 = q1 + s * tm_v[row, pl.ds(16, 16)]
        return (q0, q1)

    z16 = jnp.zeros((LANES,), jnp.float32)
    q0, q1 = lax.fori_loop(0, D_IN // LANES, qstep, (z16, z16))
    qs = tuple(q0[d] for d in range(16)) + tuple(q1[d] for d in range(16))

    iota = lax.broadcasted_iota(jnp.int32, (LANES,), 0)

    iota32 = iota * D_PC

    def process_batches(ref_c, pt_c, nbatches, carry):
        def bstep(b, c2):
            topd, topp, thr = c2
            ridx = b * (LANES * D_PC) + iota32
            acc = None
            for d in range(D_PC):
                v = plsc.load_gather(ref_c, [ridx + d])
                ad = jnp.abs(v - qs[d])
                acc = ad if acc is None else acc + ad
            pts = pt_c[pl.ds(b * LANES, LANES)]

            def merge(c3):
                sd, sp = _merge_top16(c3[0], c3[1], acc, pts)
                return (sd, sp, jnp.max(sd))

            return lax.cond(jnp.any(acc < thr), merge, lambda c3: c3, c2)

        return lax.fori_loop(0, nbatches, bstep, carry)

    carry = (jnp.full((LANES,), jnp.inf, jnp.float32),
             jnp.zeros((LANES,), jnp.float32),
             jnp.array(jnp.inf, jnp.float32))

    row0_worker = wid * ROWS_PER_WORKER

    def cstep(c, carry):
        r0 = row0_worker + c * CHUNK_ROWS
        pltpu.sync_copy(ref_flat.at[pl.ds(r0 * D_PC, CHUNK_ROWS * D_PC)],
                        ref_v0)
        pltpu.sync_copy(pt_hbm.at[pl.ds(r0, CHUNK_ROWS)], pt_v0)
        return process_batches(ref_v0, pt_v0, CHUNK_BATCHES, carry)

    carry = lax.fori_loop(0, CHUNKS, cstep, carry)

    def do_extra(carry):
        nrow = EXTRA_BATCHES * LANES
        pltpu.sync_copy(ref_flat.at[pl.ds(EXTRA_ROW0 * D_PC, nrow * D_PC)],
                        ref_v0.at[pl.ds(0, nrow * D_PC)])
        pltpu.sync_copy(pt_hbm.at[pl.ds(EXTRA_ROW0, nrow)],
                        pt_v0.at[pl.ds(0, nrow)])
        return process_batches(ref_v0, pt_v0, EXTRA_BATCHES, carry)

    carry = lax.cond(wid == NWORKERS - 1, do_extra, lambda c: c, carry)

    res_v[pl.ds(0, 16)] = carry[0]
    res_v[pl.ds(16, 16)] = carry[1]
    pltpu.sync_copy(res_v.at[pl.ds(0, 16)], outd_hbm.at[pl.ds(wid * 16, 16)])
    pltpu.sync_copy(res_v.at[pl.ds(16, 16)], outp_hbm.at[pl.ds(wid * 16, 16)])


_stage1 = functools.partial(
    pl.kernel,
    out_type=[jax.ShapeDtypeStruct((NWORKERS * 16,), jnp.float32),
              jax.ShapeDtypeStruct((NWORKERS * 16,), jnp.float32)],
    mesh=plsc.VectorSubcoreMesh(core_axis_name="c", subcore_axis_name="s"),
    compiler_params=pltpu.CompilerParams(needs_layout_passes=False),
    scratch_types=[
        pltpu.VMEM((1, D_IN), jnp.float32),
        pltpu.VMEM((D_IN, D_PC), jnp.float32),
        pltpu.VMEM((CHUNK_ROWS * D_PC,), jnp.float32),
        pltpu.VMEM((CHUNK_ROWS * D_PC,), jnp.float32),
        pltpu.VMEM((CHUNK_ROWS,), jnp.float32),
        pltpu.VMEM((CHUNK_ROWS,), jnp.float32),
        pltpu.VMEM((32,), jnp.float32),
    ],
)(_stage1_body)


def _stage2_body(d_hbm, p_hbm, out_hbm, d_v, p_v, o_v):
    cid = lax.axis_index("c")
    sid = lax.axis_index("s")

    @pl.when(jnp.logical_and(cid == 0, sid == 0))
    def _():
        pltpu.sync_copy(d_hbm, d_v)
        pltpu.sync_copy(p_hbm, p_v)

        def wstep(w, carry):
            nd = d_v[pl.ds(w * 16, 16)]
            np_ = p_v[pl.ds(w * 16, 16)]
            return _merge_top16(carry[0], carry[1], nd, np_)

        carry = (jnp.full((LANES,), jnp.inf, jnp.float32),
                 jnp.zeros((LANES,), jnp.float32))
        _, tp = lax.fori_loop(0, NWORKERS, wstep, carry)
        score = jnp.sum(tp) * (1.0 / KTOP)
        o_v[...] = jnp.full((LANES,), score, jnp.float32)
        pltpu.sync_copy(o_v, out_hbm)


_stage2 = functools.partial(
    pl.kernel,
    out_type=jax.ShapeDtypeStruct((LANES,), jnp.float32),
    mesh=plsc.VectorSubcoreMesh(core_axis_name="c", subcore_axis_name="s"),
    compiler_params=pltpu.CompilerParams(needs_layout_passes=False),
    scratch_types=[
        pltpu.VMEM((NWORKERS * 16,), jnp.float32),
        pltpu.VMEM((NWORKERS * 16,), jnp.float32),
        pltpu.VMEM((LANES,), jnp.float32),
    ],
)(_stage2_body)


def kernel(data_in, pseudotimes_arr, ref_data, transform_mat, K):
    del K
    topd, topp = _stage1(data_in, pseudotimes_arr,
                         ref_data.reshape(N_REF * D_PC), transform_mat)
    merged = _stage2(topd, topp)
    return merged[:1]


# speedup vs baseline: 1.6148x; 1.0967x over previous
import functools

import jax
import jax.numpy as jnp
from jax import lax
from jax.experimental import pallas as pl
from jax.experimental.pallas import tpu as pltpu
from jax.experimental.pallas import tpu_sc as plsc

N_REF = 1_000_000
D_IN = 128
D_PC = 32
KTOP = 16
LANES = 16
NWORKERS = 32
BATCHES_TOTAL = N_REF // LANES
BASE_BATCHES = BATCHES_TOTAL // NWORKERS
EXTRA_BATCHES = BATCHES_TOTAL - BASE_BATCHES * NWORKERS
CHUNK_BATCHES = 63
CHUNK_ROWS = CHUNK_BATCHES * LANES
CHUNKS = BASE_BATCHES // CHUNK_BATCHES
ROWS_PER_WORKER = BASE_BATCHES * LANES
EXTRA_ROW0 = BASE_BATCHES * NWORKERS * LANES
NSPLIT = 4
SUB_W = CHUNK_ROWS * D_PC // NSPLIT


def _merge_top16(td, tp, nd, np_):
    sd, sp = plsc.sort_key_val(nd, np_, descending=True)
    take = td <= sd
    ld = jnp.where(take, td, sd)
    lp = jnp.where(take, tp, sp)
    od, op = plsc.sort_key_val(ld, lp)
    return od, op


def _stage1_body(din_hbm, pt_hbm, ref_flat, tm_hbm, outd_hbm, outp_hbm,
                 din_v, tm_v, ref_v0, ref_v1, pt_v0, pt_v1, res_v,
                 sem_r0, sem_r1, sem_p0, sem_p1):
    cid = lax.axis_index("c")
    sid = lax.axis_index("s")
    wid = cid * 16 + sid

    pltpu.sync_copy(din_hbm, din_v)
    pltpu.sync_copy(tm_hbm, tm_v)

    def qstep(j, qc):
        q0, q1 = qc
        dv = din_v[0, pl.ds(j * LANES, LANES)]
        for l in range(LANES):
            s = dv[l]
            row = j * LANES + l
            q0 = q0 + s * tm_v[row, pl.ds(0, 16)]
            q1 = q1 + s * tm_v[row, pl.ds(16, 16)]
        return (q0, q1)

    z16 = jnp.zeros((LANES,), jnp.float32)
    q0, q1 = lax.fori_loop(0, D_IN // LANES, qstep, (z16, z16))
    qs = tuple(q0[d] for d in range(16)) + tuple(q1[d] for d in range(16))

    iota = lax.broadcasted_iota(jnp.int32, (LANES,), 0)
    iota32 = iota * D_PC

    def process_batches(ref_c, pt_c, nbatches, carry):
        def bstep(b, c2):
            topd, topp, thr = c2
            ridx = b * (LANES * D_PC) + iota32
            acc = None
            for d in range(D_PC):
                v = plsc.load_gather(ref_c, [ridx + d])
                ad = jnp.abs(v - qs[d])
                acc = ad if acc is None else acc + ad
            pts = pt_c[pl.ds(b * LANES, LANES)]

            def merge(c3):
                sd, sp = _merge_top16(c3[0], c3[1], acc, pts)
                return (sd, sp, jnp.max(sd))

            return lax.cond(jnp.any(acc < thr), merge, lambda c3: c3, c2)

        return lax.fori_loop(0, nbatches, bstep, carry)

    row0_worker = wid * ROWS_PER_WORKER

    def start_chunk(c, rv, pv, sem_r, sem_p):
        r0 = row0_worker + c * CHUNK_ROWS
        base = r0 * D_PC
        for q in range(NSPLIT):
            pltpu.async_copy(ref_flat.at[pl.ds(base + q * SUB_W, SUB_W)],
                             rv.at[pl.ds(q * SUB_W, SUB_W)], sem_r)
        pltpu.async_copy(pt_hbm.at[pl.ds(r0, CHUNK_ROWS)], pv, sem_p)

    def wait_chunk(rv, pv, sem_r, sem_p):
        for q in range(NSPLIT):
            pltpu.make_async_copy(ref_flat.at[pl.ds(0, SUB_W)],
                                  rv.at[pl.ds(q * SUB_W, SUB_W)],
                                  sem_r).wait()
        pltpu.make_async_copy(pt_hbm.at[pl.ds(0, CHUNK_ROWS)], pv,
                              sem_p).wait()

    carry = (jnp.full((LANES,), jnp.inf, jnp.float32),
             jnp.zeros((LANES,), jnp.float32),
             jnp.array(jnp.inf, jnp.float32))

    start_chunk(0, ref_v0, pt_v0, sem_r0, sem_p0)

    def cstep(c, carry):
        def even(carry):
            @pl.when(c + 1 < CHUNKS)
            def _():
                start_chunk(c + 1, ref_v1, pt_v1, sem_r1, sem_p1)
            wait_chunk(ref_v0, pt_v0, sem_r0, sem_p0)
            return process_batches(ref_v0, pt_v0, CHUNK_BATCHES, carry)

        def odd(carry):
            @pl.when(c + 1 < CHUNKS)
            def _():
                start_chunk(c + 1, ref_v0, pt_v0, sem_r0, sem_p0)
            wait_chunk(ref_v1, pt_v1, sem_r1, sem_p1)
            return process_batches(ref_v1, pt_v1, CHUNK_BATCHES, carry)

        return lax.cond(c % 2 == 0, even, odd, carry)

    carry = lax.fori_loop(0, CHUNKS, cstep, carry)

    def do_extra(carry):
        nrow = EXTRA_BATCHES * LANES
        pltpu.sync_copy(ref_flat.at[pl.ds(EXTRA_ROW0 * D_PC, nrow * D_PC)],
                        ref_v0.at[pl.ds(0, nrow * D_PC)])
        pltpu.sync_copy(pt_hbm.at[pl.ds(EXTRA_ROW0, nrow)],
                        pt_v0.at[pl.ds(0, nrow)])
        return process_batches(ref_v0, pt_v0, EXTRA_BATCHES, carry)

    carry = lax.cond(wid == NWORKERS - 1, do_extra, lambda c: c, carry)

    res_v[pl.ds(0, 16)] = carry[0]
    res_v[pl.ds(16, 16)] = carry[1]
    pltpu.sync_copy(res_v.at[pl.ds(0, 16)], outd_hbm.at[pl.ds(wid * 16, 16)])
    pltpu.sync_copy(res_v.at[pl.ds(16, 16)], outp_hbm.at[pl.ds(wid * 16, 16)])


_stage1 = functools.partial(
    pl.kernel,
    out_type=[jax.ShapeDtypeStruct((NWORKERS * 16,), jnp.float32),
              jax.ShapeDtypeStruct((NWORKERS * 16,), jnp.float32)],
    mesh=plsc.VectorSubcoreMesh(core_axis_name="c", subcore_axis_name="s"),
    compiler_params=pltpu.CompilerParams(needs_layout_passes=False),
    scratch_types=[
        pltpu.VMEM((1, D_IN), jnp.float32),
        pltpu.VMEM((D_IN, D_PC), jnp.float32),
        pltpu.VMEM((CHUNK_ROWS * D_PC,), jnp.float32),
        pltpu.VMEM((CHUNK_ROWS * D_PC,), jnp.float32),
        pltpu.VMEM((CHUNK_ROWS,), jnp.float32),
        pltpu.VMEM((CHUNK_ROWS,), jnp.float32),
        pltpu.VMEM((32,), jnp.float32),
        pltpu.SemaphoreType.DMA,
        pltpu.SemaphoreType.DMA,
        pltpu.SemaphoreType.DMA,
        pltpu.SemaphoreType.DMA,
    ],
)(_stage1_body)


def _merge_tc_body(d_ref, p_ref, o_ref):
    d = d_ref[...]
    p = p_ref[...]
    ii = (lax.broadcasted_iota(jnp.int32, (4, 128), 0) * 128
          + lax.broadcasted_iota(jnp.int32, (4, 128), 1))

    def step(t, carry):
        s, dd = carry
        m = jnp.min(dd)
        eq = dd == m
        idx = jnp.min(jnp.where(eq, ii, jnp.int32(1 << 30)))
        sel = ii == idx
        s = s + jnp.sum(jnp.where(sel, p, 0.0))
        dd = jnp.where(sel, jnp.inf, dd)
        return (s, dd)

    s, _ = lax.fori_loop(0, KTOP, step, (jnp.float32(0.0), d))
    o_ref[0, 0] = s * (1.0 / KTOP)


_merge_tc = pl.pallas_call(
    _merge_tc_body,
    out_shape=jax.ShapeDtypeStruct((1, 1), jnp.float32),
    out_specs=pl.BlockSpec(memory_space=pltpu.SMEM),
)


def kernel(data_in, pseudotimes_arr, ref_data, transform_mat, K):
    del K
    topd, topp = _stage1(data_in, pseudotimes_arr,
                         ref_data.reshape(N_REF * D_PC), transform_mat)
    merged = _merge_tc(topd.reshape(4, 128), topp.reshape(4, 128))
    return merged.reshape(1)


# speedup vs baseline: 2.4396x; 1.5108x over previous
import functools

import jax
import jax.numpy as jnp
from jax import lax
from jax.experimental import pallas as pl
from jax.experimental.pallas import tpu as pltpu
from jax.experimental.pallas import tpu_sc as plsc

N_REF = 1_000_000
D_IN = 128
D_PC = 32
KTOP = 16
LANES = 16
NWORKERS = 32
BLK = 8192
NBLK = (N_REF + BLK - 1) // BLK
N_PAD = NBLK * BLK
WORDS_PER_TILE = N_PAD // NWORKERS
TILE_BATCHES = WORDS_PER_TILE // LANES


def _dist_body(din_ref, tm_ref, ref_ref, o_ref):
    q = jnp.dot(din_ref[...], tm_ref[...],
                preferred_element_type=jnp.float32)
    x = ref_ref[...]
    d = jnp.sum(jnp.abs(x - q), axis=1)
    rows = pl.program_id(0) * BLK + lax.broadcasted_iota(jnp.int32, (BLK,), 0)
    o_ref[...] = jnp.where(rows < N_REF, d, jnp.inf)


_dist_tc = pl.pallas_call(
    _dist_body,
    grid=(NBLK,),
    in_specs=[
        pl.BlockSpec((1, D_IN), lambda b: (0, 0)),
        pl.BlockSpec((D_IN, D_PC), lambda b: (0, 0)),
        pl.BlockSpec((BLK, D_PC), lambda b: (b, 0)),
    ],
    out_specs=pl.BlockSpec((BLK,), lambda b: (b,)),
    out_shape=jax.ShapeDtypeStruct((N_PAD,), jnp.float32),
)


def _merge_top16(td, tp, nd, np_):
    sd, sp = plsc.sort_key_val(nd, np_, descending=True)
    take = td <= sd
    ld = jnp.where(take, td, sd)
    lp = jnp.where(take, tp, sp)
    od, op = plsc.sort_key_val(ld, lp)
    return od, op


def _topk_body(d_hbm, p_hbm, outd_hbm, outp_hbm, d_v, p_v, res_v):
    cid = lax.axis_index("c")
    sid = lax.axis_index("s")
    wid = cid * 16 + sid
    base = wid * WORDS_PER_TILE

    pltpu.sync_copy(d_hbm.at[pl.ds(base, WORDS_PER_TILE)], d_v)
    pltpu.sync_copy(p_hbm.at[pl.ds(base, WORDS_PER_TILE)], p_v)

    def bstep(b, c2):
        topd, topp, thr = c2
        dv = d_v[pl.ds(b * LANES, LANES)]

        def merge(c3):
            pv = p_v[pl.ds(b * LANES, LANES)]
            sd, sp = _merge_top16(c3[0], c3[1], dv, pv)
            return (sd, sp, jnp.max(sd))

        return lax.cond(jnp.any(dv < thr), merge, lambda c3: c3, c2)

    carry = (jnp.full((LANES,), jnp.inf, jnp.float32),
             jnp.zeros((LANES,), jnp.float32),
             jnp.array(jnp.inf, jnp.float32))
    carry = lax.fori_loop(0, TILE_BATCHES, bstep, carry)

    res_v[pl.ds(0, 16)] = carry[0]
    res_v[pl.ds(16, 16)] = carry[1]
    pltpu.sync_copy(res_v.at[pl.ds(0, 16)], outd_hbm.at[pl.ds(wid * 16, 16)])
    pltpu.sync_copy(res_v.at[pl.ds(16, 16)], outp_hbm.at[pl.ds(wid * 16, 16)])


_topk_sc = functools.partial(
    pl.kernel,
    out_type=[jax.ShapeDtypeStruct((NWORKERS * 16,), jnp.float32),
              jax.ShapeDtypeStruct((NWORKERS * 16,), jnp.float32)],
    mesh=plsc.VectorSubcoreMesh(core_axis_name="c", subcore_axis_name="s"),
    compiler_params=pltpu.CompilerParams(needs_layout_passes=False),
    scratch_types=[
        pltpu.VMEM((WORDS_PER_TILE,), jnp.float32),
        pltpu.VMEM((WORDS_PER_TILE,), jnp.float32),
        pltpu.VMEM((32,), jnp.float32),
    ],
)(_topk_body)


def _merge_tc_body(d_ref, p_ref, o_ref):
    d = d_ref[...]
    p = p_ref[...]
    ii = (lax.broadcasted_iota(jnp.int32, (4, 128), 0) * 128
          + lax.broadcasted_iota(jnp.int32, (4, 128), 1))

    def step(t, carry):
        s, dd = carry
        m = jnp.min(dd)
        eq = dd == m
        idx = jnp.min(jnp.where(eq, ii, jnp.int32(1 << 30)))
        sel = ii == idx
        s = s + jnp.sum(jnp.where(sel, p, 0.0))
        dd = jnp.where(sel, jnp.inf, dd)
        return (s, dd)

    s, _ = lax.fori_loop(0, KTOP, step, (jnp.float32(0.0), d))
    o_ref[0, 0] = s * (1.0 / KTOP)


_merge_tc = pl.pallas_call(
    _merge_tc_body,
    out_shape=jax.ShapeDtypeStruct((1, 1), jnp.float32),
    out_specs=pl.BlockSpec(memory_space=pltpu.SMEM),
)


def kernel(data_in, pseudotimes_arr, ref_data, transform_mat, K):
    del K
    dists = _dist_tc(data_in, transform_mat, ref_data)
    pts_p = jnp.pad(pseudotimes_arr, (0, N_PAD - N_REF))
    topd, topp = _topk_sc(dists, pts_p)
    merged = _merge_tc(topd.reshape(4, 128), topp.reshape(4, 128))
    return merged.reshape(1)
